# baseline (speedup 1.0000x reference)
import functools

import jax
import jax.numpy as jnp
from jax import lax
from jax.experimental import pallas as pl
from jax.experimental.pallas import tpu as pltpu
from jax.experimental.pallas import tpu_sc as plsc

_N = 10000
_E = 320000
_D = 128
_H1 = 128
_H2 = 64
_H3 = 64
_L = 32
_G = 64

_NC = 2
_NS = 16
_NW = _NC * _NS
_EW = _E // _NW
_B = 128
_NBATCH = 80
_EPAD = _NBATCH * _B - _EW
_NBATCH_CS = 160
_EPAD_CS = _NS * _NBATCH_CS * _B - _E
_CHUNK = 632

_BLK = 1000

_sc_mesh = plsc.VectorSubcoreMesh(
    core_axis_name="c", subcore_axis_name="s", num_cores=_NC, num_subcores=_NS
)


def _make_deg_kernel():
  @functools.partial(
      pl.kernel,
      mesh=_sc_mesh,
      compiler_params=pltpu.CompilerParams(use_tc_tiling_on_sc=False),
      out_type=jax.ShapeDtypeStruct((_NC, _N, 16), jnp.float32),
      scratch_types=[
          pltpu.VMEM((_NBATCH, _B), jnp.int32),
          pltpu.VMEM((_B, 16), jnp.float32),
          pltpu.VMEM_SHARED((_N + 8, 16), jnp.float32),
      ],
  )
  def deg_kernel(dstp_hbm, ones_hbm, zeros_hbm, out_hbm, dst_v, ones_v, acc_sh):
    c = lax.axis_index("c")
    s = lax.axis_index("s")
    w = c * _NS + s
    off = pl.multiple_of(jnp.minimum(s * _CHUNK, _N - _CHUNK), 8)
    pltpu.sync_copy(zeros_hbm, acc_sh.at[pl.ds(off, _CHUNK)])
    pltpu.sync_copy(ones_hbm, ones_v)
    pltpu.sync_copy(dstp_hbm.at[w], dst_v)
    plsc.subcore_barrier()

    @pl.loop(0, _NBATCH)
    def _(b):
      pltpu.sync_copy(ones_v, acc_sh.at[dst_v.at[b]], add=True)

    plsc.subcore_barrier()
    pltpu.sync_copy(acc_sh.at[pl.ds(off, _CHUNK)],
                    out_hbm.at[c, pl.ds(off, _CHUNK)])

  return deg_kernel


def _make_agg_kernel(nb, colsplit):
  nstage = nb // _NBATCH
  _R = 8

  @functools.partial(
      pl.kernel,
      mesh=_sc_mesh,
      compiler_params=pltpu.CompilerParams(use_tc_tiling_on_sc=False),
      out_type=jax.ShapeDtypeStruct((_NC, _N, _H2), jnp.float32),
      scratch_types=[
          pltpu.VMEM((_NBATCH, _B), jnp.int32),
          pltpu.VMEM((_NBATCH, _B), jnp.int32),
          [pltpu.VMEM((_B, _H2), jnp.float32)] * _R,
          pltpu.VMEM_SHARED((_N + 8, _H2), jnp.float32),
          [pltpu.SemaphoreType.DMA] * _R,
          [pltpu.SemaphoreType.DMA] * _R,
      ],
  )
  def agg_kernel(table_hbm, srcp_hbm, dstp_hbm, zeros_hbm, out_hbm,
                 src_v, dst_v, rows, acc_sh, gsem, ssem):
    c = lax.axis_index("c")
    s = lax.axis_index("s")
    off = pl.multiple_of(jnp.minimum(s * _CHUNK, _N - _CHUNK), 8)
    pltpu.sync_copy(zeros_hbm, acc_sh.at[pl.ds(off, _CHUNK)])
    if colsplit:
      w = s
      tab = table_hbm.at[c]
    else:
      w = c * _NS + s
      tab = table_hbm
    plsc.subcore_barrier()

    def start_g(b, k):
      pltpu.async_copy(tab.at[src_v.at[b]], rows[k], gsem[k])

    def wait_g(b, k):
      pltpu.make_async_copy(tab.at[src_v.at[b]], rows[k], gsem[k]).wait()

    def start_s(b, k):
      pltpu.async_copy(rows[k], acc_sh.at[dst_v.at[b]], ssem[k], add=True)

    def wait_s(b, k):
      pltpu.make_async_copy(rows[k], acc_sh.at[dst_v.at[b]], ssem[k]).wait()

    nbs = _NBATCH
    for stage in range(nstage):
      pltpu.sync_copy(srcp_hbm.at[w, pl.ds(stage * nbs, nbs)], src_v)
      pltpu.sync_copy(dstp_hbm.at[w, pl.ds(stage * nbs, nbs)], dst_v)
      start_g(0, 0)
      start_g(1, 1)
      for b in range(_R - 2):
        wait_g(b, b % _R)
        start_s(b, b % _R)
        start_g(b + 2, (b + 2) % _R)

      @pl.loop(0, (nbs - _R) // _R)
      def _(i):
        b0 = (_R - 2) + _R * i
        for j in range(_R):
          k = (_R - 2 + j) % _R
          b = b0 + j
          wait_g(b, k)
          start_s(b, k)
          wait_s(b - (_R - 2), (j + 0) % _R)
          start_g(b + 2, (j + 0) % _R)

      wait_g(nbs - 2, (_R - 2) % _R)
      start_s(nbs - 2, (_R - 2) % _R)
      wait_g(nbs - 1, (_R - 1) % _R)
      start_s(nbs - 1, (_R - 1) % _R)
      for j in range(_R):
        wait_s(nbs - _R + j, (nbs - _R + j) % _R)

    plsc.subcore_barrier()
    pltpu.sync_copy(acc_sh.at[pl.ds(off, _CHUNK)],
                    out_hbm.at[c, pl.ds(off, _CHUNK)])

  return agg_kernel


_deg_call = _make_deg_kernel()
_agg_call_cs = _make_agg_kernel(_NBATCH_CS, True)
_agg_call_es = _make_agg_kernel(_NBATCH, False)


def _isd_from_deg(deg_ref):
  d = deg_ref[0, :, 0:1] + deg_ref[1, :, 0:1] + 1.0
  return lax.rsqrt(d)


def _mm1_body(feat_ref, w_ref, deg_ref, out_ref):
  isd = _isd_from_deg(deg_ref)
  hw = jnp.dot(feat_ref[...], w_ref[...],
               preferred_element_type=jnp.float32) * isd
  out_ref[0, :, :] = hw[:, :_H2]
  out_ref[1, :, :] = hw[:, _H2:]


def _comb_mm_body(agg_ref, hwp_ref, deg_ref, w_ref, out_ref):
  isd = _isd_from_deg(deg_ref)
  full = (agg_ref[...] + hwp_ref[...])
  h = jnp.maximum(
      jnp.concatenate([full[0], full[1]], axis=1) * isd, 0.0)
  out_ref[...] = jnp.dot(h, w_ref[...],
                         preferred_element_type=jnp.float32) * isd


def _pool_body(agg_ref, hwp_ref, deg_ref, seg_ref, sums_ref, counts_ref):
  i = pl.program_id(0)
  isd = _isd_from_deg(deg_ref)
  h2 = jnp.maximum((agg_ref[0, :, :] + agg_ref[1, :, :] + hwp_ref[...]) * isd,
                   0.0)
  seg = seg_ref[0, 0, :]
  onehot = (lax.broadcasted_iota(jnp.int32, (_G, _BLK), 0)
            == seg[None, :]).astype(jnp.float32)
  part = jnp.dot(onehot, h2, preferred_element_type=jnp.float32)
  cnt = jnp.sum(onehot, axis=1, keepdims=True) * jnp.ones((1, _H2),
                                                          jnp.float32)

  @pl.when(i == 0)
  def _():
    sums_ref[...] = jnp.zeros_like(sums_ref)
    counts_ref[...] = jnp.zeros_like(counts_ref)

  sums_ref[...] += part
  counts_ref[...] += cnt


def _head_body(sums_ref, counts_ref, w3_ref, b3_ref, w4_ref, b4_ref, out_ref):
  pooled = sums_ref[...] / jnp.maximum(counts_ref[...], 1.0)
  h3 = jnp.maximum(
      jnp.dot(pooled, w3_ref[...], preferred_element_type=jnp.float32)
      + b3_ref[...], 0.0)
  out_ref[...] = (jnp.dot(h3, w4_ref[...], preferred_element_type=jnp.float32)
                  + b4_ref[...])


def kernel(features, edge_index, segment_ids, W1, W2, W3, b3, W4, b4):
  src = edge_index[0]
  dst = edge_index[1]

  pad = jnp.zeros((_NW, _EPAD), jnp.int32)
  srcp = jnp.concatenate([src.reshape(_NW, _EW), pad],
                         axis=1).reshape(_NW, _NBATCH, _B)
  dstp = jnp.concatenate([dst.reshape(_NW, _EW), pad + _N],
                         axis=1).reshape(_NW, _NBATCH, _B)
  pad_cs = jnp.zeros((_NS, _EPAD_CS // _NS), jnp.int32)
  srcq = jnp.concatenate([src.reshape(_NS, _E // _NS), pad_cs],
                         axis=1).reshape(_NS, _NBATCH_CS, _B)
  dstq = jnp.concatenate([dst.reshape(_NS, _E // _NS), pad_cs + _N],
                         axis=1).reshape(_NS, _NBATCH_CS, _B)

  ones16 = jnp.ones((_B, 16), jnp.float32)
  zeros16 = jnp.zeros((_CHUNK, 16), jnp.float32)
  zeros64 = jnp.zeros((_CHUNK, _H2), jnp.float32)

  deg16 = _deg_call(dstp, ones16, zeros16)

  grid = (_N // _BLK,)
  deg_spec = pl.BlockSpec((_NC, _BLK, 16), lambda i: (0, i, 0))

  hwp1h = pl.pallas_call(
      _mm1_body,
      grid=grid,
      in_specs=[
          pl.BlockSpec((_BLK, _D), lambda i: (i, 0)),
          pl.BlockSpec((_D, _H1), lambda i: (0, 0)),
          deg_spec,
      ],
      out_specs=pl.BlockSpec((_NC, _BLK, _H2), lambda i: (0, i, 0)),
      out_shape=jax.ShapeDtypeStruct((_NC, _N, _H2), jnp.float32),
  )(features, W1, deg16)

  agg1 = _agg_call_cs(hwp1h, srcq, dstq, zeros64)

  hwp2 = pl.pallas_call(
      _comb_mm_body,
      grid=grid,
      in_specs=[
          pl.BlockSpec((_NC, _BLK, _H2), lambda i: (0, i, 0)),
          pl.BlockSpec((_NC, _BLK, _H2), lambda i: (0, i, 0)),
          deg_spec,
          pl.BlockSpec((_H1, _H2), lambda i: (0, 0)),
      ],
      out_specs=pl.BlockSpec((_BLK, _H2), lambda i: (i, 0)),
      out_shape=jax.ShapeDtypeStruct((_N, _H2), jnp.float32),
  )(agg1, hwp1h, deg16, W2)

  agg2 = _agg_call_es(hwp2, srcp, dstp, zeros64)

  seg3d = segment_ids.reshape(_N // _BLK, 1, _BLK)
  sums, counts = pl.pallas_call(
      _pool_body,
      grid=grid,
      in_specs=[
          pl.BlockSpec((_NC, _BLK, _H2), lambda i: (0, i, 0)),
          pl.BlockSpec((_BLK, _H2), lambda i: (i, 0)),
          deg_spec,
          pl.BlockSpec((1, 1, _BLK), lambda i: (i, 0, 0)),
      ],
      out_specs=[
          pl.BlockSpec((_G, _H2), lambda i: (0, 0)),
          pl.BlockSpec((_G, _H2), lambda i: (0, 0)),
      ],
      out_shape=[
          jax.ShapeDtypeStruct((_G, _H2), jnp.float32),
          jax.ShapeDtypeStruct((_G, _H2), jnp.float32),
      ],
  )(agg2, hwp2, deg16, seg3d)

  out = pl.pallas_call(
      _head_body,
      in_specs=[
          pl.BlockSpec((_G, _H2), lambda: (0, 0)),
          pl.BlockSpec((_G, _H2), lambda: (0, 0)),
          pl.BlockSpec((_H2, _H3), lambda: (0, 0)),
          pl.BlockSpec((1, _H3), lambda: (0, 0)),
          pl.BlockSpec((_H3, _L), lambda: (0, 0)),
          pl.BlockSpec((1, _L), lambda: (0, 0)),
      ],
      out_specs=pl.BlockSpec((_G, _L), lambda: (0, 0)),
      out_shape=jax.ShapeDtypeStruct((_G, _L), jnp.float32),
  )(sums, counts, W3, b3.reshape(1, _H3), W4, b4.reshape(1, _L))

  return out


# speedup vs baseline: 1.0955x; 1.0955x over previous
import functools

import jax
import jax.numpy as jnp
from jax import lax
from jax.experimental import pallas as pl
from jax.experimental.pallas import tpu as pltpu
from jax.experimental.pallas import tpu_sc as plsc

_N = 10000
_E = 320000
_D = 128
_H1 = 128
_H2 = 64
_H3 = 64
_L = 32
_G = 64

_NC = 2
_NS = 16
_NW = _NC * _NS
_EW = _E // _NW
_B = 128
_NBATCH = 80
_EPAD = _NBATCH * _B - _EW
_NBATCH_CS = 160
_EPAD_CS = _NS * _NBATCH_CS * _B - _E
_CHUNK = 632

_BLK = 1000

_sc_mesh = plsc.VectorSubcoreMesh(
    core_axis_name="c", subcore_axis_name="s", num_cores=_NC, num_subcores=_NS
)


def _make_deg_kernel():
  @functools.partial(
      pl.kernel,
      mesh=_sc_mesh,
      compiler_params=pltpu.CompilerParams(use_tc_tiling_on_sc=False),
      out_type=jax.ShapeDtypeStruct((_NC, _N, 16), jnp.float32),
      scratch_types=[
          pltpu.VMEM((_NBATCH, _B), jnp.int32),
          pltpu.VMEM((_B, 16), jnp.float32),
          pltpu.VMEM_SHARED((_N + 8, 16), jnp.float32),
      ],
  )
  def deg_kernel(dstp_hbm, ones_hbm, zeros_hbm, out_hbm, dst_v, ones_v, acc_sh):
    c = lax.axis_index("c")
    s = lax.axis_index("s")
    w = c * _NS + s
    off = pl.multiple_of(jnp.minimum(s * _CHUNK, _N - _CHUNK), 8)
    pltpu.sync_copy(zeros_hbm, acc_sh.at[pl.ds(off, _CHUNK)])
    pltpu.sync_copy(ones_hbm, ones_v)
    pltpu.sync_copy(dstp_hbm.at[w], dst_v)
    plsc.subcore_barrier()

    @pl.loop(0, _NBATCH)
    def _(b):
      pltpu.sync_copy(ones_v, acc_sh.at[dst_v.at[b]], add=True)

    plsc.subcore_barrier()
    pltpu.sync_copy(acc_sh.at[pl.ds(off, _CHUNK)],
                    out_hbm.at[c, pl.ds(off, _CHUNK)])

  return deg_kernel


def _make_agg_kernel(hcols):
  nb = _NBATCH_CS
  nstage = nb // _NBATCH
  _R = 8

  @functools.partial(
      pl.kernel,
      mesh=_sc_mesh,
      compiler_params=pltpu.CompilerParams(use_tc_tiling_on_sc=False),
      out_type=jax.ShapeDtypeStruct((_NC, _N, hcols), jnp.float32),
      scratch_types=[
          pltpu.VMEM((_NBATCH, _B), jnp.int32),
          pltpu.VMEM((_NBATCH, _B), jnp.int32),
          [pltpu.VMEM((_B, hcols), jnp.float32)] * _R,
          pltpu.VMEM_SHARED((_N + 8, hcols), jnp.float32),
          [pltpu.SemaphoreType.DMA] * _R,
          [pltpu.SemaphoreType.DMA] * _R,
      ],
  )
  def agg_kernel(table_hbm, srcp_hbm, dstp_hbm, zeros_hbm, out_hbm,
                 src_v, dst_v, rows, acc_sh, gsem, ssem):
    c = lax.axis_index("c")
    s = lax.axis_index("s")
    w = s
    tab = table_hbm.at[c]
    off = pl.multiple_of(jnp.minimum(s * _CHUNK, _N - _CHUNK), 8)
    pltpu.sync_copy(zeros_hbm, acc_sh.at[pl.ds(off, _CHUNK)])
    plsc.subcore_barrier()

    def start_g(b, k):
      pltpu.async_copy(tab.at[src_v.at[b]], rows[k], gsem[k])

    def wait_g(b, k):
      pltpu.make_async_copy(tab.at[src_v.at[b]], rows[k], gsem[k]).wait()

    def start_s(b, k):
      pltpu.async_copy(rows[k], acc_sh.at[dst_v.at[b]], ssem[k], add=True)

    def wait_s(b, k):
      pltpu.make_async_copy(rows[k], acc_sh.at[dst_v.at[b]], ssem[k]).wait()

    nbs = _NBATCH
    for stage in range(nstage):
      pltpu.sync_copy(srcp_hbm.at[w, pl.ds(stage * nbs, nbs)], src_v)
      pltpu.sync_copy(dstp_hbm.at[w, pl.ds(stage * nbs, nbs)], dst_v)
      start_g(0, 0)
      start_g(1, 1)
      for b in range(_R - 2):
        wait_g(b, b % _R)
        start_s(b, b % _R)
        start_g(b + 2, (b + 2) % _R)

      @pl.loop(0, (nbs - _R) // _R)
      def _(i):
        b0 = (_R - 2) + _R * i
        for j in range(_R):
          k = (_R - 2 + j) % _R
          b = b0 + j
          wait_g(b, k)
          start_s(b, k)
          wait_s(b - (_R - 2), (j + 0) % _R)
          start_g(b + 2, (j + 0) % _R)

      wait_g(nbs - 2, (_R - 2) % _R)
      start_s(nbs - 2, (_R - 2) % _R)
      wait_g(nbs - 1, (_R - 1) % _R)
      start_s(nbs - 1, (_R - 1) % _R)
      for j in range(_R):
        wait_s(nbs - _R + j, (nbs - _R + j) % _R)

    plsc.subcore_barrier()
    pltpu.sync_copy(acc_sh.at[pl.ds(off, _CHUNK)],
                    out_hbm.at[c, pl.ds(off, _CHUNK)])

  return agg_kernel


_deg_call = _make_deg_kernel()
_agg_call_l1 = _make_agg_kernel(_H1 // 2)
_agg_call_l2 = _make_agg_kernel(_H2 // 2)


def _isd_from_deg(deg_ref):
  d = deg_ref[0, :, 0:1] + deg_ref[1, :, 0:1] + 1.0
  return lax.rsqrt(d)


def _mm1_body(feat_ref, w_ref, deg_ref, out_ref):
  isd = _isd_from_deg(deg_ref)
  hw = jnp.dot(feat_ref[...], w_ref[...],
               preferred_element_type=jnp.float32) * isd
  out_ref[0, :, :] = hw[:, :_H2]
  out_ref[1, :, :] = hw[:, _H2:]


def _comb_mm_body(agg_ref, hwp_ref, deg_ref, w_ref, out_ref):
  isd = _isd_from_deg(deg_ref)
  full = (agg_ref[...] + hwp_ref[...])
  h = jnp.maximum(
      jnp.concatenate([full[0], full[1]], axis=1) * isd, 0.0)
  hw = jnp.dot(h, w_ref[...], preferred_element_type=jnp.float32) * isd
  out_ref[0, :, :] = hw[:, :_H2 // 2]
  out_ref[1, :, :] = hw[:, _H2 // 2:]


def _pool_body(agg_ref, hwp_ref, deg_ref, seg_ref, sums_ref, counts_ref):
  i = pl.program_id(0)
  isd = _isd_from_deg(deg_ref)
  full = agg_ref[...] + hwp_ref[...]
  h2 = jnp.maximum(
      jnp.concatenate([full[0], full[1]], axis=1) * isd, 0.0)
  seg = seg_ref[0, 0, :]
  onehot = (lax.broadcasted_iota(jnp.int32, (_G, _BLK), 0)
            == seg[None, :]).astype(jnp.float32)
  part = jnp.dot(onehot, h2, preferred_element_type=jnp.float32)
  cnt = jnp.sum(onehot, axis=1, keepdims=True) * jnp.ones((1, _H2),
                                                          jnp.float32)

  @pl.when(i == 0)
  def _():
    sums_ref[...] = jnp.zeros_like(sums_ref)
    counts_ref[...] = jnp.zeros_like(counts_ref)

  sums_ref[...] += part
  counts_ref[...] += cnt


def _head_body(sums_ref, counts_ref, w3_ref, b3_ref, w4_ref, b4_ref, out_ref):
  pooled = sums_ref[...] / jnp.maximum(counts_ref[...], 1.0)
  h3 = jnp.maximum(
      jnp.dot(pooled, w3_ref[...], preferred_element_type=jnp.float32)
      + b3_ref[...], 0.0)
  out_ref[...] = (jnp.dot(h3, w4_ref[...], preferred_element_type=jnp.float32)
                  + b4_ref[...])


def kernel(features, edge_index, segment_ids, W1, W2, W3, b3, W4, b4):
  src = edge_index[0]
  dst = edge_index[1]

  pad = jnp.zeros((_NW, _EPAD), jnp.int32)
  dstp = jnp.concatenate([dst.reshape(_NW, _EW), pad + _N],
                         axis=1).reshape(_NW, _NBATCH, _B)
  pad_cs = jnp.zeros((_NS, _EPAD_CS // _NS), jnp.int32)
  srcq = jnp.concatenate([src.reshape(_NS, _E // _NS), pad_cs],
                         axis=1).reshape(_NS, _NBATCH_CS, _B)
  dstq = jnp.concatenate([dst.reshape(_NS, _E // _NS), pad_cs + _N],
                         axis=1).reshape(_NS, _NBATCH_CS, _B)

  ones16 = jnp.ones((_B, 16), jnp.float32)
  zeros16 = jnp.zeros((_CHUNK, 16), jnp.float32)
  zeros64 = jnp.zeros((_CHUNK, _H2), jnp.float32)
  zeros32 = jnp.zeros((_CHUNK, _H2 // 2), jnp.float32)

  deg16 = _deg_call(dstp, ones16, zeros16)

  grid = (_N // _BLK,)
  deg_spec = pl.BlockSpec((_NC, _BLK, 16), lambda i: (0, i, 0))

  hwp1h = pl.pallas_call(
      _mm1_body,
      grid=grid,
      in_specs=[
          pl.BlockSpec((_BLK, _D), lambda i: (i, 0)),
          pl.BlockSpec((_D, _H1), lambda i: (0, 0)),
          deg_spec,
      ],
      out_specs=pl.BlockSpec((_NC, _BLK, _H2), lambda i: (0, i, 0)),
      out_shape=jax.ShapeDtypeStruct((_NC, _N, _H2), jnp.float32),
  )(features, W1, deg16)

  agg1 = _agg_call_l1(hwp1h, srcq, dstq, zeros64)

  hwp2h = pl.pallas_call(
      _comb_mm_body,
      grid=grid,
      in_specs=[
          pl.BlockSpec((_NC, _BLK, _H2), lambda i: (0, i, 0)),
          pl.BlockSpec((_NC, _BLK, _H2), lambda i: (0, i, 0)),
          deg_spec,
          pl.BlockSpec((_H1, _H2), lambda i: (0, 0)),
      ],
      out_specs=pl.BlockSpec((_NC, _BLK, _H2 // 2), lambda i: (0, i, 0)),
      out_shape=jax.ShapeDtypeStruct((_NC, _N, _H2 // 2), jnp.float32),
  )(agg1, hwp1h, deg16, W2)

  agg2 = _agg_call_l2(hwp2h, srcq, dstq, zeros32)

  seg3d = segment_ids.reshape(_N // _BLK, 1, _BLK)
  sums, counts = pl.pallas_call(
      _pool_body,
      grid=grid,
      in_specs=[
          pl.BlockSpec((_NC, _BLK, _H2 // 2), lambda i: (0, i, 0)),
          pl.BlockSpec((_NC, _BLK, _H2 // 2), lambda i: (0, i, 0)),
          deg_spec,
          pl.BlockSpec((1, 1, _BLK), lambda i: (i, 0, 0)),
      ],
      out_specs=[
          pl.BlockSpec((_G, _H2), lambda i: (0, 0)),
          pl.BlockSpec((_G, _H2), lambda i: (0, 0)),
      ],
      out_shape=[
          jax.ShapeDtypeStruct((_G, _H2), jnp.float32),
          jax.ShapeDtypeStruct((_G, _H2), jnp.float32),
      ],
  )(agg2, hwp2h, deg16, seg3d)

  out = pl.pallas_call(
      _head_body,
      in_specs=[
          pl.BlockSpec((_G, _H2), lambda: (0, 0)),
          pl.BlockSpec((_G, _H2), lambda: (0, 0)),
          pl.BlockSpec((_H2, _H3), lambda: (0, 0)),
          pl.BlockSpec((1, _H3), lambda: (0, 0)),
          pl.BlockSpec((_H3, _L), lambda: (0, 0)),
          pl.BlockSpec((1, _L), lambda: (0, 0)),
      ],
      out_specs=pl.BlockSpec((_G, _L), lambda: (0, 0)),
      out_shape=jax.ShapeDtypeStruct((_G, _L), jnp.float32),
  )(sums, counts, W3, b3.reshape(1, _H3), W4, b4.reshape(1, _L))

  return out


# speedup vs baseline: 1.1016x; 1.0055x over previous
import functools

import jax
import jax.numpy as jnp
from jax import lax
from jax.experimental import pallas as pl
from jax.experimental.pallas import tpu as pltpu
from jax.experimental.pallas import tpu_sc as plsc

_N = 10000
_E = 320000
_D = 128
_H1 = 128
_H2 = 64
_H3 = 64
_L = 32
_G = 64

_NC = 2
_NS = 16
_NW = _NC * _NS
_EW = _E // _NW
_B = 128
_NBATCH = 80
_EPAD = _NBATCH * _B - _EW
_NBATCH_CS = 160
_EPAD_CS = _NS * _NBATCH_CS * _B - _E
_CHUNK = 632

_BLK = 1000

_sc_mesh = plsc.VectorSubcoreMesh(
    core_axis_name="c", subcore_axis_name="s", num_cores=_NC, num_subcores=_NS
)


def _make_deg_kernel():
  @functools.partial(
      pl.kernel,
      mesh=_sc_mesh,
      compiler_params=pltpu.CompilerParams(use_tc_tiling_on_sc=False),
      out_type=jax.ShapeDtypeStruct((_NC, _N, 16), jnp.float32),
      scratch_types=[
          pltpu.VMEM((_NBATCH, _B), jnp.int32),
          pltpu.VMEM((_B, 16), jnp.float32),
          pltpu.VMEM_SHARED((_N + 8, 16), jnp.float32),
      ],
  )
  def deg_kernel(dstp_hbm, ones_hbm, zeros_hbm, out_hbm, dst_v, ones_v, acc_sh):
    c = lax.axis_index("c")
    s = lax.axis_index("s")
    w = c * _NS + s
    off = pl.multiple_of(jnp.minimum(s * _CHUNK, _N - _CHUNK), 8)
    pltpu.sync_copy(zeros_hbm, acc_sh.at[pl.ds(off, _CHUNK)])
    pltpu.sync_copy(ones_hbm, ones_v)
    pltpu.sync_copy(dstp_hbm.at[w], dst_v)
    plsc.subcore_barrier()

    @pl.loop(0, _NBATCH)
    def _(b):
      pltpu.sync_copy(ones_v, acc_sh.at[dst_v.at[b]], add=True)

    plsc.subcore_barrier()
    pltpu.sync_copy(acc_sh.at[pl.ds(off, _CHUNK)],
                    out_hbm.at[c, pl.ds(off, _CHUNK)])

  return deg_kernel


def _make_agg_kernel(hcols):
  nb = _NBATCH_CS
  nbs = _NBATCH if hcols > _H2 // 2 else _NBATCH_CS
  nstage = nb // nbs
  _R = 8

  @functools.partial(
      pl.kernel,
      mesh=_sc_mesh,
      compiler_params=pltpu.CompilerParams(use_tc_tiling_on_sc=False),
      out_type=jax.ShapeDtypeStruct((_NC, _N, hcols), jnp.float32),
      scratch_types=[
          pltpu.VMEM((nbs, _B), jnp.int32),
          pltpu.VMEM((nbs, _B), jnp.int32),
          [pltpu.VMEM((_B, hcols), jnp.float32)] * _R,
          pltpu.VMEM_SHARED((_N + 8, hcols), jnp.float32),
          [pltpu.SemaphoreType.DMA] * _R,
          [pltpu.SemaphoreType.DMA] * _R,
      ],
  )
  def agg_kernel(table_hbm, srcp_hbm, dstp_hbm, zeros_hbm, out_hbm,
                 src_v, dst_v, rows, acc_sh, gsem, ssem):
    c = lax.axis_index("c")
    s = lax.axis_index("s")
    w = s
    tab = table_hbm.at[c]
    off = pl.multiple_of(jnp.minimum(s * _CHUNK, _N - _CHUNK), 8)
    pltpu.sync_copy(zeros_hbm, acc_sh.at[pl.ds(off, _CHUNK)])
    plsc.subcore_barrier()

    def start_g(b, k):
      pltpu.async_copy(tab.at[src_v.at[b]], rows[k], gsem[k])

    def wait_g(b, k):
      pltpu.make_async_copy(tab.at[src_v.at[b]], rows[k], gsem[k]).wait()

    def start_s(b, k):
      pltpu.async_copy(rows[k], acc_sh.at[dst_v.at[b]], ssem[k], add=True)

    def wait_s(b, k):
      pltpu.make_async_copy(rows[k], acc_sh.at[dst_v.at[b]], ssem[k]).wait()

    for stage in range(nstage):
      pltpu.sync_copy(srcp_hbm.at[w, pl.ds(stage * nbs, nbs)], src_v)
      pltpu.sync_copy(dstp_hbm.at[w, pl.ds(stage * nbs, nbs)], dst_v)
      start_g(0, 0)
      start_g(1, 1)
      for b in range(_R - 2):
        wait_g(b, b % _R)
        start_s(b, b % _R)
        start_g(b + 2, (b + 2) % _R)

      @pl.loop(0, (nbs - _R) // _R)
      def _(i):
        b0 = (_R - 2) + _R * i
        for j in range(_R):
          k = (_R - 2 + j) % _R
          b = b0 + j
          wait_g(b, k)
          start_s(b, k)
          wait_s(b - (_R - 2), (j + 0) % _R)
          start_g(b + 2, (j + 0) % _R)

      wait_g(nbs - 2, (_R - 2) % _R)
      start_s(nbs - 2, (_R - 2) % _R)
      wait_g(nbs - 1, (_R - 1) % _R)
      start_s(nbs - 1, (_R - 1) % _R)
      for j in range(_R):
        wait_s(nbs - _R + j, (nbs - _R + j) % _R)

    plsc.subcore_barrier()
    pltpu.sync_copy(acc_sh.at[pl.ds(off, _CHUNK)],
                    out_hbm.at[c, pl.ds(off, _CHUNK)])

  return agg_kernel


_deg_call = _make_deg_kernel()
_agg_call_l1 = _make_agg_kernel(_H1 // 2)
_agg_call_l2 = _make_agg_kernel(_H2 // 2)


def _isd_from_deg(deg_ref):
  d = deg_ref[0, :, 0:1] + deg_ref[1, :, 0:1] + 1.0
  return lax.rsqrt(d)


def _mm1_body(feat_ref, w_ref, deg_ref, out_ref):
  isd = _isd_from_deg(deg_ref)
  hw = jnp.dot(feat_ref[...], w_ref[...],
               preferred_element_type=jnp.float32) * isd
  out_ref[0, :, :] = hw[:, :_H2]
  out_ref[1, :, :] = hw[:, _H2:]


def _comb_mm_body(agg_ref, hwp_ref, deg_ref, w_ref, out_ref):
  isd = _isd_from_deg(deg_ref)
  full = (agg_ref[...] + hwp_ref[...])
  h = jnp.maximum(
      jnp.concatenate([full[0], full[1]], axis=1) * isd, 0.0)
  hw = jnp.dot(h, w_ref[...], preferred_element_type=jnp.float32) * isd
  out_ref[0, :, :] = hw[:, :_H2 // 2]
  out_ref[1, :, :] = hw[:, _H2 // 2:]


def _pool_head_body(agg_ref, hwp_ref, deg_ref, seg_ref, w3_ref, b3_ref,
                    w4_ref, b4_ref, out_ref, sums_ref, counts_ref):
  i = pl.program_id(0)
  isd = _isd_from_deg(deg_ref)
  full = agg_ref[...] + hwp_ref[...]
  h2 = jnp.maximum(
      jnp.concatenate([full[0], full[1]], axis=1) * isd, 0.0)
  seg = seg_ref[0, 0, :]
  onehot = (lax.broadcasted_iota(jnp.int32, (_G, _BLK), 0)
            == seg[None, :]).astype(jnp.float32)
  part = jnp.dot(onehot, h2, preferred_element_type=jnp.float32)
  cnt = jnp.sum(onehot, axis=1, keepdims=True) * jnp.ones((1, _H2),
                                                          jnp.float32)

  @pl.when(i == 0)
  def _():
    sums_ref[...] = jnp.zeros_like(sums_ref)
    counts_ref[...] = jnp.zeros_like(counts_ref)

  sums_ref[...] += part
  counts_ref[...] += cnt

  @pl.when(i == _N // _BLK - 1)
  def _():
    pooled = sums_ref[...] / jnp.maximum(counts_ref[...], 1.0)
    h3 = jnp.maximum(
        jnp.dot(pooled, w3_ref[...], preferred_element_type=jnp.float32)
        + b3_ref[...], 0.0)
    out_ref[...] = (jnp.dot(h3, w4_ref[...],
                            preferred_element_type=jnp.float32) + b4_ref[...])


def kernel(features, edge_index, segment_ids, W1, W2, W3, b3, W4, b4):
  src = edge_index[0]
  dst = edge_index[1]

  pad = jnp.zeros((_NW, _EPAD), jnp.int32)
  dstp = jnp.concatenate([dst.reshape(_NW, _EW), pad + _N],
                         axis=1).reshape(_NW, _NBATCH, _B)
  pad_cs = jnp.zeros((_NS, _EPAD_CS // _NS), jnp.int32)
  srcq = jnp.concatenate([src.reshape(_NS, _E // _NS), pad_cs],
                         axis=1).reshape(_NS, _NBATCH_CS, _B)
  dstq = jnp.concatenate([dst.reshape(_NS, _E // _NS), pad_cs + _N],
                         axis=1).reshape(_NS, _NBATCH_CS, _B)

  ones16 = jnp.ones((_B, 16), jnp.float32)
  zeros16 = jnp.zeros((_CHUNK, 16), jnp.float32)
  zeros64 = jnp.zeros((_CHUNK, _H2), jnp.float32)
  zeros32 = jnp.zeros((_CHUNK, _H2 // 2), jnp.float32)

  deg16 = _deg_call(dstp, ones16, zeros16)

  grid = (_N // _BLK,)
  deg_spec = pl.BlockSpec((_NC, _BLK, 16), lambda i: (0, i, 0))

  hwp1h = pl.pallas_call(
      _mm1_body,
      grid=grid,
      in_specs=[
          pl.BlockSpec((_BLK, _D), lambda i: (i, 0)),
          pl.BlockSpec((_D, _H1), lambda i: (0, 0)),
          deg_spec,
      ],
      out_specs=pl.BlockSpec((_NC, _BLK, _H2), lambda i: (0, i, 0)),
      out_shape=jax.ShapeDtypeStruct((_NC, _N, _H2), jnp.float32),
  )(features, W1, deg16)

  agg1 = _agg_call_l1(hwp1h, srcq, dstq, zeros64)

  hwp2h = pl.pallas_call(
      _comb_mm_body,
      grid=grid,
      in_specs=[
          pl.BlockSpec((_NC, _BLK, _H2), lambda i: (0, i, 0)),
          pl.BlockSpec((_NC, _BLK, _H2), lambda i: (0, i, 0)),
          deg_spec,
          pl.BlockSpec((_H1, _H2), lambda i: (0, 0)),
      ],
      out_specs=pl.BlockSpec((_NC, _BLK, _H2 // 2), lambda i: (0, i, 0)),
      out_shape=jax.ShapeDtypeStruct((_NC, _N, _H2 // 2), jnp.float32),
  )(agg1, hwp1h, deg16, W2)

  agg2 = _agg_call_l2(hwp2h, srcq, dstq, zeros32)

  seg3d = segment_ids.reshape(_N // _BLK, 1, _BLK)
  out = pl.pallas_call(
      _pool_head_body,
      grid=grid,
      in_specs=[
          pl.BlockSpec((_NC, _BLK, _H2 // 2), lambda i: (0, i, 0)),
          pl.BlockSpec((_NC, _BLK, _H2 // 2), lambda i: (0, i, 0)),
          deg_spec,
          pl.BlockSpec((1, 1, _BLK), lambda i: (i, 0, 0)),
          pl.BlockSpec((_H2, _H3), lambda i: (0, 0)),
          pl.BlockSpec((1, _H3), lambda i: (0, 0)),
          pl.BlockSpec((_H3, _L), lambda i: (0, 0)),
          pl.BlockSpec((1, _L), lambda i: (0, 0)),
      ],
      out_specs=pl.BlockSpec((_G, _L), lambda i: (0, 0)),
      out_shape=jax.ShapeDtypeStruct((_G, _L), jnp.float32),
      scratch_shapes=[
          pltpu.VMEM((_G, _H2), jnp.float32),
          pltpu.VMEM((_G, _H2), jnp.float32),
      ],
  )(agg2, hwp2h, deg16, seg3d, W3, b3.reshape(1, _H3), W4,
    b4.reshape(1, _L))

  return out
